# Initial kernel scaffold; baseline (speedup 1.0000x reference)
import functools

import jax
import jax.numpy as jnp
from jax import lax
from jax.experimental import pallas as pl
from jax.experimental.pallas import tpu as pltpu
from jax.experimental.pallas import tpu_sc as plsc

N = 10000
E = 320000
D_IN = 128
D_H = 128
N_CLASSES = 16

NPAD = 10240
ROWS_PER_TILE = NPAD // 16
EDGES_PER_CORE = E // 2
EDGES_PER_TILE = EDGES_PER_CORE // 16
K = 80
NBATCH = EDGES_PER_TILE // K

BR = 400
GRID = N // BR


def _make_sc_edge_kernel(D):
    CH = D // 16
    mesh = plsc.VectorSubcoreMesh(core_axis_name="c", subcore_axis_name="s")

    @functools.partial(
        pl.kernel,
        out_type=[
            jax.ShapeDtypeStruct((2, NPAD, D), jnp.float32),
            jax.ShapeDtypeStruct((2, NPAD), jnp.float32),
        ],
        mesh=mesh,
        scratch_types=[
            pltpu.VMEM((N,), jnp.float32),
            pltpu.VMEM((N,), jnp.float32),
            pltpu.VMEM((K,), jnp.int32),
            pltpu.VMEM((K,), jnp.int32),
            pltpu.VMEM((K,), jnp.float32),
            pltpu.VMEM((K, D), jnp.float32),
            pltpu.VMEM_SHARED((NPAD, D), jnp.float32),
            pltpu.VMEM_SHARED((NPAD,), jnp.float32),
            pltpu.SemaphoreType.DMA,
        ],
    )
    def sc_edge(h_hbm, el_hbm, er_hbm, src_hbm, dst_hbm, u_out, s_out,
                el_v, er_v, sidx, didx, x_v, rows, u_sh, s_sh, sem):
        cid = lax.axis_index("c")
        tid = lax.axis_index("s")
        zero16 = jnp.zeros((16,), jnp.float32)

        @pl.loop(0, K)
        def _(k):
            for c in range(CH):
                rows[k, pl.ds(c * 16, 16)] = zero16
        for i in range(K // 16):
            x_v[pl.ds(i * 16, 16)] = zero16
        row0 = tid * ROWS_PER_TILE
        for r in range(ROWS_PER_TILE // K):
            pltpu.sync_copy(rows, u_sh.at[pl.ds(row0 + r * K, K)])
            pltpu.sync_copy(x_v, s_sh.at[pl.ds(row0 + r * K, K)])

        pltpu.sync_copy(el_hbm, el_v)
        pltpu.sync_copy(er_hbm, er_v)
        plsc.subcore_barrier()

        base = cid * EDGES_PER_CORE + tid * EDGES_PER_TILE

        @pl.loop(0, NBATCH)
        def _(bt):
            off = base + bt * K
            pltpu.sync_copy(src_hbm.at[pl.ds(off, K)], sidx)
            pltpu.sync_copy(dst_hbm.at[pl.ds(off, K)], didx)
            gather = pltpu.async_copy(h_hbm.at[sidx], rows, sem)
            for i in range(K // 16):
                s16 = sidx[pl.ds(i * 16, 16)]
                d16 = didx[pl.ds(i * 16, 16)]
                e = plsc.load_gather(el_v, [s16]) + plsc.load_gather(er_v, [d16])
                e = jnp.where(e >= 0.0, e, 0.2 * e)
                x_v[pl.ds(i * 16, 16)] = jnp.exp(e)
            gather.wait()

            @pl.loop(0, K)
            def _(k):
                xk = plsc.load_gather(x_v, [jnp.broadcast_to(k, (16,))])
                for c in range(CH):
                    rows[k, pl.ds(c * 16, 16)] = rows[k, pl.ds(c * 16, 16)] * xk

            pltpu.sync_copy(rows, u_sh.at[didx], add=True)
            pltpu.sync_copy(x_v, s_sh.at[didx], add=True)

        plsc.subcore_barrier()
        pltpu.sync_copy(u_sh.at[pl.ds(row0, ROWS_PER_TILE)],
                        u_out.at[cid, pl.ds(row0, ROWS_PER_TILE)])
        pltpu.sync_copy(s_sh.at[pl.ds(row0, ROWS_PER_TILE)],
                        s_out.at[cid, pl.ds(row0, ROWS_PER_TILE)])

    return sc_edge


_sc_edge_128 = _make_sc_edge_kernel(D_H)
_sc_edge_16 = _make_sc_edge_kernel(N_CLASSES)


def _tc_first(x, W, al, ar):
    D = W.shape[1]

    def body(x_ref, w_ref, al_ref, ar_ref, h_ref, el_ref, er_ref):
        h = jnp.dot(x_ref[...], w_ref[...], preferred_element_type=jnp.float32)
        h_ref[...] = h
        el_ref[...] = jnp.dot(h, al_ref[...], preferred_element_type=jnp.float32)
        er_ref[...] = jnp.dot(h, ar_ref[...], preferred_element_type=jnp.float32)

    return pl.pallas_call(
        body,
        grid=(GRID,),
        in_specs=[
            pl.BlockSpec((BR, x.shape[1]), lambda i: (i, 0)),
            pl.BlockSpec(W.shape, lambda i: (0, 0)),
            pl.BlockSpec((D, 1), lambda i: (0, 0)),
            pl.BlockSpec((D, 1), lambda i: (0, 0)),
        ],
        out_specs=[
            pl.BlockSpec((BR, D), lambda i: (i, 0)),
            pl.BlockSpec((BR, 1), lambda i: (i, 0)),
            pl.BlockSpec((BR, 1), lambda i: (i, 0)),
        ],
        out_shape=[
            jax.ShapeDtypeStruct((N, D), jnp.float32),
            jax.ShapeDtypeStruct((N, 1), jnp.float32),
            jax.ShapeDtypeStruct((N, 1), jnp.float32),
        ],
    )(x, W, al.reshape(-1, 1), ar.reshape(-1, 1))


def _tc_mid(u2, s2, b_prev, W, al, ar):
    Din = u2.shape[-1]
    D = W.shape[1]

    def body(u_ref, s_ref, b_ref, w_ref, al_ref, ar_ref, h_ref, el_ref, er_ref):
        u = u_ref[0] + u_ref[1]
        s = s_ref[0] + s_ref[1]
        act = jnp.maximum(u / (s + 1e-9) + b_ref[...], 0.0)
        h = jnp.dot(act, w_ref[...], preferred_element_type=jnp.float32)
        h_ref[...] = h
        el_ref[...] = jnp.dot(h, al_ref[...], preferred_element_type=jnp.float32)
        er_ref[...] = jnp.dot(h, ar_ref[...], preferred_element_type=jnp.float32)

    return pl.pallas_call(
        body,
        grid=(GRID,),
        in_specs=[
            pl.BlockSpec((2, BR, Din), lambda i: (0, i, 0)),
            pl.BlockSpec((2, BR, 1), lambda i: (0, i, 0)),
            pl.BlockSpec((1, Din), lambda i: (0, 0)),
            pl.BlockSpec(W.shape, lambda i: (0, 0)),
            pl.BlockSpec((D, 1), lambda i: (0, 0)),
            pl.BlockSpec((D, 1), lambda i: (0, 0)),
        ],
        out_specs=[
            pl.BlockSpec((BR, D), lambda i: (i, 0)),
            pl.BlockSpec((BR, 1), lambda i: (i, 0)),
            pl.BlockSpec((BR, 1), lambda i: (i, 0)),
        ],
        out_shape=[
            jax.ShapeDtypeStruct((N, D), jnp.float32),
            jax.ShapeDtypeStruct((N, 1), jnp.float32),
            jax.ShapeDtypeStruct((N, 1), jnp.float32),
        ],
    )(u2, s2, b_prev, W, al.reshape(-1, 1), ar.reshape(-1, 1))


def _tc_final(u2, s2, b):
    D = u2.shape[-1]

    def body(u_ref, s_ref, b_ref, o_ref):
        u = u_ref[0] + u_ref[1]
        s = s_ref[0] + s_ref[1]
        o_ref[...] = jnp.maximum(u / (s + 1e-9) + b_ref[...], 0.0)

    return pl.pallas_call(
        body,
        grid=(GRID,),
        in_specs=[
            pl.BlockSpec((2, BR, D), lambda i: (0, i, 0)),
            pl.BlockSpec((2, BR, 1), lambda i: (0, i, 0)),
            pl.BlockSpec((1, D), lambda i: (0, 0)),
        ],
        out_specs=pl.BlockSpec((BR, D), lambda i: (i, 0)),
        out_shape=jax.ShapeDtypeStruct((N, D), jnp.float32),
    )(u2, s2, b)


def kernel(in_feat, g, W1, al1, ar1, b1, W2, al2, ar2, b2, W3, al3, ar3, b3):
    src = g[0]
    dst = g[1]

    h, el, er = _tc_first(in_feat, W1, al1, ar1)
    u, s = _sc_edge_128(h, el.reshape(N), er.reshape(N), src, dst)
    h, el, er = _tc_mid(u, s.reshape(2, NPAD, 1), b1.reshape(1, -1), W2, al2, ar2)
    u, s = _sc_edge_128(h, el.reshape(N), er.reshape(N), src, dst)
    h, el, er = _tc_mid(u, s.reshape(2, NPAD, 1), b2.reshape(1, -1), W3, al3, ar3)
    u, s = _sc_edge_16(h, el.reshape(N), er.reshape(N), src, dst)
    out = _tc_final(u, s.reshape(2, NPAD, 1), b3.reshape(1, -1))
    return out


# speedup vs baseline: 20.5949x; 20.5949x over previous
import dataclasses
import functools

import jax
import jax.numpy as jnp
from jax import lax
from jax.experimental import pallas as pl
from jax.experimental.pallas import tpu as pltpu
from jax.experimental.pallas import tpu_sc as plsc

N = 10000
E = 320000
D_IN = 128
D_H = 128
N_CLASSES = 16

NPAD = 10240
ROWS_PER_TILE = NPAD // 16
EDGES_PER_CORE = E // 2
EDGES_PER_TILE = EDGES_PER_CORE // 16
K = 80
NBATCH = EDGES_PER_TILE // K

BR = 400
GRID = N // BR


def _make_sc_edge_kernel(D):
    CH = D // 16
    mesh = plsc.VectorSubcoreMesh(core_axis_name="c", subcore_axis_name="s")
    cp = pltpu.CompilerParams()
    if "needs_layout_passes" in pltpu.CompilerParams.__dataclass_fields__:
        cp = dataclasses.replace(cp, needs_layout_passes=False)

    @functools.partial(
        pl.kernel,
        compiler_params=cp,
        out_type=[
            jax.ShapeDtypeStruct((2, NPAD, D), jnp.float32),
            jax.ShapeDtypeStruct((2, NPAD), jnp.float32),
        ],
        mesh=mesh,
        scratch_types=[
            pltpu.VMEM((N,), jnp.float32),
            pltpu.VMEM((N,), jnp.float32),
            pltpu.VMEM((K,), jnp.int32),
            pltpu.VMEM((K,), jnp.int32),
            pltpu.VMEM((K,), jnp.float32),
            pltpu.VMEM((K, D), jnp.float32),
            pltpu.VMEM_SHARED((NPAD, D), jnp.float32),
            pltpu.VMEM_SHARED((NPAD,), jnp.float32),
            pltpu.SemaphoreType.DMA,
        ],
    )
    def sc_edge(h_hbm, el_hbm, er_hbm, src_hbm, dst_hbm, u_out, s_out,
                el_v, er_v, sidx, didx, x_v, rows, u_sh, s_sh, sem):
        cid = lax.axis_index("c")
        tid = lax.axis_index("s")
        zero16 = jnp.zeros((16,), jnp.float32)

        @pl.loop(0, K)
        def _(k):
            for c in range(CH):
                rows[k, pl.ds(c * 16, 16)] = zero16
        for i in range(K // 16):
            x_v[pl.ds(i * 16, 16)] = zero16
        row0 = tid * ROWS_PER_TILE
        for r in range(ROWS_PER_TILE // K):
            pltpu.sync_copy(rows, u_sh.at[pl.ds(row0 + r * K, K)])
            pltpu.sync_copy(x_v, s_sh.at[pl.ds(row0 + r * K, K)])

        pltpu.sync_copy(el_hbm, el_v)
        pltpu.sync_copy(er_hbm, er_v)
        plsc.subcore_barrier()

        base = cid * EDGES_PER_CORE + tid * EDGES_PER_TILE

        @pl.loop(0, NBATCH)
        def _(bt):
            off = base + bt * K
            pltpu.sync_copy(src_hbm.at[pl.ds(off, K)], sidx)
            pltpu.sync_copy(dst_hbm.at[pl.ds(off, K)], didx)
            gather = pltpu.async_copy(h_hbm.at[sidx], rows, sem)
            for i in range(K // 16):
                s16 = sidx[pl.ds(i * 16, 16)]
                d16 = didx[pl.ds(i * 16, 16)]
                e = plsc.load_gather(el_v, [s16]) + plsc.load_gather(er_v, [d16])
                e = jnp.where(e >= 0.0, e, 0.2 * e)
                x_v[pl.ds(i * 16, 16)] = jnp.exp(e)
            gather.wait()

            @pl.loop(0, K)
            def _(k):
                xk = plsc.load_gather(x_v, [jnp.broadcast_to(k, (16,))])
                for c in range(CH):
                    rows[k, pl.ds(c * 16, 16)] = rows[k, pl.ds(c * 16, 16)] * xk

            pltpu.sync_copy(rows, u_sh.at[didx], add=True)
            pltpu.sync_copy(x_v, s_sh.at[didx], add=True)

        plsc.subcore_barrier()
        pltpu.sync_copy(u_sh.at[pl.ds(row0, ROWS_PER_TILE)],
                        u_out.at[cid, pl.ds(row0, ROWS_PER_TILE)])
        pltpu.sync_copy(s_sh.at[pl.ds(row0, ROWS_PER_TILE)],
                        s_out.at[cid, pl.ds(row0, ROWS_PER_TILE)])

    return sc_edge


_sc_edge_128 = _make_sc_edge_kernel(D_H)


def _tc_first(x, W, al, ar):
    D = W.shape[1]

    def body(x_ref, w_ref, al_ref, ar_ref, h_ref, el_ref, er_ref):
        h = jnp.dot(x_ref[...], w_ref[...], preferred_element_type=jnp.float32)
        h_ref[...] = h
        el_ref[...] = jnp.dot(h, al_ref[...], preferred_element_type=jnp.float32)
        er_ref[...] = jnp.dot(h, ar_ref[...], preferred_element_type=jnp.float32)

    return pl.pallas_call(
        body,
        grid=(GRID,),
        in_specs=[
            pl.BlockSpec((BR, x.shape[1]), lambda i: (i, 0)),
            pl.BlockSpec(W.shape, lambda i: (0, 0)),
            pl.BlockSpec((D, 1), lambda i: (0, 0)),
            pl.BlockSpec((D, 1), lambda i: (0, 0)),
        ],
        out_specs=[
            pl.BlockSpec((BR, D), lambda i: (i, 0)),
            pl.BlockSpec((BR, 1), lambda i: (i, 0)),
            pl.BlockSpec((BR, 1), lambda i: (i, 0)),
        ],
        out_shape=[
            jax.ShapeDtypeStruct((N, D), jnp.float32),
            jax.ShapeDtypeStruct((N, 1), jnp.float32),
            jax.ShapeDtypeStruct((N, 1), jnp.float32),
        ],
    )(x, W, al.reshape(-1, 1), ar.reshape(-1, 1))


def _tc_mid(u2, s2, b_prev, W, al, ar):
    Din = u2.shape[-1]
    D = W.shape[1]

    def body(u_ref, s_ref, b_ref, w_ref, al_ref, ar_ref, h_ref, el_ref, er_ref):
        u = u_ref[0] + u_ref[1]
        s = s_ref[0] + s_ref[1]
        act = jnp.maximum(u / (s + 1e-9) + b_ref[...], 0.0)
        h = jnp.dot(act, w_ref[...], preferred_element_type=jnp.float32)
        h_ref[...] = h
        el_ref[...] = jnp.dot(h, al_ref[...], preferred_element_type=jnp.float32)
        er_ref[...] = jnp.dot(h, ar_ref[...], preferred_element_type=jnp.float32)

    return pl.pallas_call(
        body,
        grid=(GRID,),
        in_specs=[
            pl.BlockSpec((2, BR, Din), lambda i: (0, i, 0)),
            pl.BlockSpec((2, BR, 1), lambda i: (0, i, 0)),
            pl.BlockSpec((1, Din), lambda i: (0, 0)),
            pl.BlockSpec(W.shape, lambda i: (0, 0)),
            pl.BlockSpec((D, 1), lambda i: (0, 0)),
            pl.BlockSpec((D, 1), lambda i: (0, 0)),
        ],
        out_specs=[
            pl.BlockSpec((BR, D), lambda i: (i, 0)),
            pl.BlockSpec((BR, 1), lambda i: (i, 0)),
            pl.BlockSpec((BR, 1), lambda i: (i, 0)),
        ],
        out_shape=[
            jax.ShapeDtypeStruct((N, D), jnp.float32),
            jax.ShapeDtypeStruct((N, 1), jnp.float32),
            jax.ShapeDtypeStruct((N, 1), jnp.float32),
        ],
    )(u2, s2, b_prev, W, al.reshape(-1, 1), ar.reshape(-1, 1))


def _tc_final(u2, s2, b):
    D = b.shape[-1]
    Dfull = u2.shape[-1]

    def body(u_ref, s_ref, b_ref, o_ref):
        u = u_ref[0, :, :D] + u_ref[1, :, :D]
        s = s_ref[0] + s_ref[1]
        o_ref[...] = jnp.maximum(u / (s + 1e-9) + b_ref[...], 0.0)

    return pl.pallas_call(
        body,
        grid=(GRID,),
        in_specs=[
            pl.BlockSpec((2, BR, Dfull), lambda i: (0, i, 0)),
            pl.BlockSpec((2, BR, 1), lambda i: (0, i, 0)),
            pl.BlockSpec((1, D), lambda i: (0, 0)),
        ],
        out_specs=pl.BlockSpec((BR, D), lambda i: (i, 0)),
        out_shape=jax.ShapeDtypeStruct((N, D), jnp.float32),
    )(u2, s2, b)


def kernel(in_feat, g, W1, al1, ar1, b1, W2, al2, ar2, b2, W3, al3, ar3, b3):
    src = g[0]
    dst = g[1]
    W3p = jnp.pad(W3, ((0, 0), (0, D_H - N_CLASSES)))
    al3p = jnp.pad(al3, (0, D_H - N_CLASSES))
    ar3p = jnp.pad(ar3, (0, D_H - N_CLASSES))

    h, el, er = _tc_first(in_feat, W1, al1, ar1)
    u, s = _sc_edge_128(h, el.reshape(N), er.reshape(N), src, dst)
    h, el, er = _tc_mid(u, s.reshape(2, NPAD, 1), b1.reshape(1, -1), W2, al2, ar2)
    u, s = _sc_edge_128(h, el.reshape(N), er.reshape(N), src, dst)
    h, el, er = _tc_mid(u, s.reshape(2, NPAD, 1), b2.reshape(1, -1), W3p, al3p, ar3p)
    u, s = _sc_edge_128(h, el.reshape(N), er.reshape(N), src, dst)
    out = _tc_final(u, s.reshape(2, NPAD, 1), b3.reshape(1, -1))
    return out
